# baseline (speedup 1.0000x reference)
import functools

import jax
import jax.numpy as jnp
from jax import lax
from jax.experimental import pallas as pl
from jax.experimental.pallas import tpu as pltpu
from jax.experimental.pallas import tpu_sc as plsc

B = 4096 * 20
D = 1000
NC = 2
NS = 16
NW = NC * NS
ROWS_PER_W = B // NW
CB = 64
NCH = ROWS_PER_W // CB
L = 16


@functools.partial(
    pl.kernel,
    mesh=plsc.VectorSubcoreMesh(core_axis_name="c", subcore_axis_name="s"),
    compiler_params=pltpu.CompilerParams(needs_layout_passes=False),
    out_type=jax.ShapeDtypeStruct((B * D,), jnp.float32),
    scratch_types=[
        pltpu.VMEM((ROWS_PER_W,), jnp.int32),
        pltpu.VMEM((CB * D,), jnp.float32),
        pltpu.VMEM((CB * D,), jnp.float32),
        pltpu.SemaphoreType.DMA,
        pltpu.SemaphoreType.DMA,
    ],
)
def _onehot_sc(x_hbm, z_hbm, out_hbm, idx_v, buf0, buf1, sem0, sem1):
    cid = lax.axis_index("c")
    sid = lax.axis_index("s")
    wid = sid * NC + cid
    row0 = wid * ROWS_PER_W
    pltpu.sync_copy(x_hbm.at[pl.ds(row0, ROWS_PER_W)], idx_v)
    pltpu.sync_copy(z_hbm, buf0)
    pltpu.sync_copy(z_hbm, buf1)

    lanes = lax.iota(jnp.int32, L)
    ones = jnp.full((L,), 1.0, jnp.float32)
    zeros = jnp.zeros((L,), jnp.float32)
    rowbase = [(j * L + lanes) * D for j in range(CB // L)]

    def scatter(c, buf, val):
        for j in range(CB // L):
            cols = idx_v[pl.ds(c * CB + j * L, L)]
            plsc.store_scatter(buf, [rowbase[j] + cols], val)

    def start(c, buf, sem):
        pltpu.make_async_copy(
            buf, out_hbm.at[pl.ds((row0 + c * CB) * D, CB * D)], sem
        ).start()

    def wait(c, buf, sem):
        pltpu.make_async_copy(
            buf, out_hbm.at[pl.ds((row0 + c * CB) * D, CB * D)], sem
        ).wait()

    bufs = (buf0, buf1)
    sems = (sem0, sem1)

    for b in range(2):
        scatter(b, bufs[b], ones)
        start(b, bufs[b], sems[b])

    def ring_body(g, carry):
        for b in range(2):
            c = g * 2 + b
            wait(c - 2, bufs[b], sems[b])
            scatter(c - 2, bufs[b], zeros)
            scatter(c, bufs[b], ones)
            start(c, bufs[b], sems[b])
        return carry

    lax.fori_loop(1, NCH // 2, ring_body, 0)

    for b in range(2):
        wait(NCH - 2 + b, bufs[b], sems[b])


def kernel(x, eye):
    del eye
    xf = x.reshape(-1)
    zeros = jnp.zeros((CB * D,), jnp.float32)
    out = _onehot_sc(xf, zeros)
    return out.reshape(x.shape[0], x.shape[1], D)


# speedup vs baseline: 1.4900x; 1.4900x over previous
import functools

import jax
import jax.numpy as jnp
from jax import lax
from jax.experimental import pallas as pl
from jax.experimental.pallas import tpu as pltpu
from jax.experimental.pallas import tpu_sc as plsc

M0 = 4096
R = 20
D = 1000
NC = 2
NS = 16
NW = NC * NS
M_PER_W = M0 // NW
CM = 4
NCH = M_PER_W // CM
CR = CM * R
L = 16


@functools.partial(
    pl.kernel,
    mesh=plsc.VectorSubcoreMesh(core_axis_name="c", subcore_axis_name="s"),
    compiler_params=pltpu.CompilerParams(needs_layout_passes=False),
    out_type=jax.ShapeDtypeStruct((M0, R, D), jnp.float32),
    scratch_types=[
        pltpu.VMEM((M_PER_W * R,), jnp.int32),
        pltpu.VMEM((CM, R, D), jnp.float32),
    ],
)
def _onehot_sc(x_hbm, z_hbm, out_hbm, idx_v, buf_v):
    cid = lax.axis_index("c")
    sid = lax.axis_index("s")
    wid = sid * NC + cid
    m0 = wid * M_PER_W
    pltpu.sync_copy(x_hbm.at[pl.ds(m0 * R, M_PER_W * R)], idx_v)
    pltpu.sync_copy(z_hbm, buf_v)

    lanes = lax.iota(jnp.int32, L)
    ones = jnp.full((L,), 1.0, jnp.float32)
    zeros = jnp.zeros((L,), jnp.float32)
    mi = [((g * L + lanes) // R) for g in range(CR // L)]
    ri = [((g * L + lanes) % R) for g in range(CR // L)]

    def scatter(c, val):
        for g in range(CR // L):
            cols = idx_v[pl.ds(c * CR + g * L, L)]
            plsc.store_scatter(buf_v, [mi[g], ri[g], cols], val)

    def chunk_body(c, carry):
        scatter(c, ones)
        pltpu.sync_copy(buf_v, out_hbm.at[pl.ds(m0 + c * CM, CM)])
        scatter(c, zeros)
        return carry

    lax.fori_loop(0, NCH, chunk_body, 0)


def kernel(x, eye):
    del eye
    xf = x.reshape(-1)
    zeros = jnp.zeros((CM, R, D), jnp.float32)
    return _onehot_sc(xf, zeros)


# speedup vs baseline: 4.9917x; 3.3501x over previous
import functools

import jax
import jax.numpy as jnp
from jax import lax
from jax.experimental import pallas as pl
from jax.experimental.pallas import tpu as pltpu
from jax.experimental.pallas import tpu_sc as plsc

M = 4096
R = 20
D = 1000
HA = 496
HB = D - HA
NC = 2
NS = 16
NW = NC * NS
MW = M // NW
L = 16
NG = MW // L


@functools.partial(
    pl.kernel,
    mesh=plsc.VectorSubcoreMesh(core_axis_name="c", subcore_axis_name="s"),
    compiler_params=pltpu.CompilerParams(needs_layout_passes=False),
    out_type=jax.ShapeDtypeStruct((R, D, M), jnp.float32),
    scratch_types=[
        pltpu.VMEM((R, MW), jnp.int32),
        pltpu.VMEM((HA, MW), jnp.float32),
        pltpu.VMEM((HB, MW), jnp.float32),
        pltpu.SemaphoreType.DMA,
        pltpu.SemaphoreType.DMA,
    ],
)
def _onehot_sc(xt_hbm, z_hbm, out_hbm, idx_v, buf_a, buf_b, sem_a, sem_b):
    cid = lax.axis_index("c")
    sid = lax.axis_index("s")
    wid = sid * NC + cid
    m0 = wid * MW
    pltpu.sync_copy(xt_hbm.at[:, pl.ds(m0, MW)], idx_v)
    pltpu.sync_copy(z_hbm.at[pl.ds(0, HA)], buf_a)
    pltpu.sync_copy(z_hbm, buf_b)

    lanes = lax.iota(jnp.int32, L)
    ones = jnp.full((L,), 1.0, jnp.float32)
    zeros = jnp.zeros((L,), jnp.float32)

    def scatter(r, buf, lo, hi, val):
        for g in range(NG):
            cols = idx_v[r, pl.ds(g * L, L)] - lo
            mask = (cols >= 0) & (cols < hi - lo)
            plsc.store_scatter(buf, [cols, g * L + lanes], val, mask=mask)

    def start(r, lo, hi, buf, sem):
        pltpu.make_async_copy(
            buf, out_hbm.at[r, pl.ds(lo, hi - lo), pl.ds(m0, MW)], sem
        ).start()

    def wait(r, lo, hi, buf, sem):
        pltpu.make_async_copy(
            buf, out_hbm.at[r, pl.ds(lo, hi - lo), pl.ds(m0, MW)], sem
        ).wait()

    scatter(0, buf_a, 0, HA, ones)
    start(0, 0, HA, buf_a, sem_a)
    scatter(0, buf_b, HA, D, ones)
    start(0, HA, D, buf_b, sem_b)

    def slab_body(r, carry):
        wait(r - 1, 0, HA, buf_a, sem_a)
        scatter(r - 1, buf_a, 0, HA, zeros)
        scatter(r, buf_a, 0, HA, ones)
        start(r, 0, HA, buf_a, sem_a)
        wait(r - 1, HA, D, buf_b, sem_b)
        scatter(r - 1, buf_b, HA, D, zeros)
        scatter(r, buf_b, HA, D, ones)
        start(r, HA, D, buf_b, sem_b)
        return carry

    lax.fori_loop(1, R, slab_body, 0)
    wait(R - 1, 0, HA, buf_a, sem_a)
    wait(R - 1, HA, D, buf_b, sem_b)


def kernel(x, eye):
    del eye
    xt = jnp.transpose(x)
    zeros = jnp.zeros((HB, MW), jnp.float32)
    out = _onehot_sc(xt, zeros)
    return jnp.transpose(out, (2, 0, 1))


# speedup vs baseline: 5.0975x; 1.0212x over previous
import functools

import jax
import jax.numpy as jnp
from jax import lax
from jax.experimental import pallas as pl
from jax.experimental.pallas import tpu as pltpu
from jax.experimental.pallas import tpu_sc as plsc

M = 4096
R = 20
D = 1000
CA = 496
CB = D - CA
NC = 2
NS = 16
NW = NC * NS
S = 2
NP = NW // S
MWS = M // NP
RG = R // S
L = 16
NG = MWS // L


@functools.partial(
    pl.kernel,
    mesh=plsc.VectorSubcoreMesh(core_axis_name="c", subcore_axis_name="s"),
    compiler_params=pltpu.CompilerParams(needs_layout_passes=False),
    out_type=jax.ShapeDtypeStruct((R, D, M), jnp.float32),
    scratch_types=[
        pltpu.VMEM((MWS,), jnp.int32),
        pltpu.VMEM((CB, MWS), jnp.float32),
    ],
)
def _onehot_sc(xt_hbm, z_hbm, out_hbm, idx_v, buf_v):
    cid = lax.axis_index("c")
    sid = lax.axis_index("s")
    wid = sid * NC + cid
    p = wid // S
    h = wid % S
    m0 = p * MWS
    r0 = h * RG
    pltpu.sync_copy(z_hbm, buf_v)

    lanes = lax.iota(jnp.int32, L)
    ones = jnp.full((L,), 1.0, jnp.float32)
    zeros = jnp.zeros((L,), jnp.float32)

    def scatter(lo, hi, val):
        for g in range(NG):
            cols = idx_v[pl.ds(g * L, L)] - lo
            mask = (cols >= 0) & (cols < hi - lo)
            plsc.store_scatter(buf_v, [cols, g * L + lanes], val, mask=mask)

    def slab_body(r, carry):
        pltpu.sync_copy(xt_hbm.at[r, pl.ds(m0, MWS)], idx_v)
        for lo, hi in ((0, CA), (CA, D)):
            scatter(lo, hi, ones)
            pltpu.sync_copy(
                buf_v.at[pl.ds(0, hi - lo)],
                out_hbm.at[r, pl.ds(lo, hi - lo), pl.ds(m0, MWS)],
            )
            scatter(lo, hi, zeros)
        return carry

    lax.fori_loop(r0, r0 + RG, slab_body, 0)


def kernel(x, eye):
    del eye
    xt = jnp.transpose(x)
    zeros = jnp.zeros((CB, MWS), jnp.float32)
    out = _onehot_sc(xt, zeros)
    return jnp.transpose(out, (2, 0, 1))
